# Initial kernel scaffold; baseline (speedup 1.0000x reference)
import jax
import jax.numpy as jnp
from jax.experimental import pallas as pl


def kernel(x, edge_index_pos, edge_index_inv, batch, type_emb, out_emb, W1, b1, W2, b2):
    raise NotImplementedError("write your pallas kernel here")


# speedup vs baseline: 2.8188x; 2.8188x over previous
import functools

import jax
import jax.numpy as jnp
from jax import lax
from jax.experimental import pallas as pl
from jax.experimental.pallas import tpu as pltpu
from jax.experimental.pallas import tpu_sc as plsc

N = 10000
E = 160000
EMB = 128
NUM_LAYER = 3
NG = 64

NS = 16
NP = 10240
RPS = NP // NS
CHUNK = 128
NCHUNKS = 80
EPS = NCHUNKS * CHUNK
EPAD = NS * EPS

_mesh = plsc.VectorSubcoreMesh(core_axis_name="c", subcore_axis_name="s")


def _seg_body(h_hbm, e_hbm, z_hbm, idx, bufa, acc, gsa):
    cid = lax.axis_index("c")
    sid = lax.axis_index("s")
    r0 = sid * RPS

    pltpu.sync_copy(h_hbm.at[pl.ds(r0, RPS)], acc.at[pl.ds(r0, RPS)])
    pltpu.sync_copy(e_hbm.at[cid, sid], idx)

    plsc.subcore_barrier()

    def body(ch, carry):
        pltpu.async_copy(h_hbm.at[idx.at[0, ch]], bufa, gsa).wait()
        pltpu.sync_copy(bufa, acc.at[idx.at[1, ch]], add=True)
        return carry

    lax.fori_loop(0, NCHUNKS, body, 0)

    plsc.subcore_barrier()
    pltpu.sync_copy(acc.at[pl.ds(r0, RPS)], z_hbm.at[cid, pl.ds(r0, RPS)])


_seg_call = pl.kernel(
    _seg_body,
    out_type=[jax.ShapeDtypeStruct((2, NP, EMB), jnp.float32)],
    mesh=_mesh,
    scratch_types=[
        pltpu.VMEM((2, NCHUNKS, CHUNK), jnp.int32),
        pltpu.VMEM((CHUNK, EMB), jnp.float32),
        pltpu.VMEM_SHARED((NP, EMB), jnp.float32),
        pltpu.SemaphoreType.DMA,
    ],
)


def _embed_body(x_ref, te_ref, oe_ref, out_ref):
    xb = x_ref[...]
    x0 = xb[:, 0:1].astype(jnp.float32)
    x1 = xb[:, 1:2].astype(jnp.float32)
    t0 = te_ref[0:1, :]
    t1 = te_ref[1:2, :]
    o0 = oe_ref[0:1, :]
    o1 = oe_ref[1:2, :]
    h = t0 + (t1 - t0) * x0 + o0 + (o1 - o0) * x1
    out_ref[:N, :] = h
    out_ref[N:, :] = jnp.zeros((NP - N, EMB), jnp.float32)


def _embed_call(x, type_emb, out_emb):
    return pl.pallas_call(
        _embed_body,
        out_shape=jax.ShapeDtypeStruct((NP, EMB), jnp.float32),
    )(x, type_emb, out_emb)


def _layer_body(zp_ref, zi_ref, w1p, b1p, w2p, b2p, w1i, b1i, w2i, b2i,
                out_ref, *, final):
    zp = zp_ref[0]
    a = jnp.maximum(
        jnp.dot(zp, w1p[...], preferred_element_type=jnp.float32) + b1p[...], 0.0)
    hp = jnp.dot(a, w2p[...], preferred_element_type=jnp.float32) + b2p[...]
    zi = zi_ref[0]
    b = jnp.maximum(
        jnp.dot(zi, w1i[...], preferred_element_type=jnp.float32) + b1i[...], 0.0)
    hi = jnp.dot(b, w2i[...], preferred_element_type=jnp.float32) + b2i[...]
    o = hp + hi
    if not final:
        o = jnp.maximum(o, 0.0)
    out_ref[...] = o


def _layer_call(z2, weights, final):
    BN = 1024
    zpspec = pl.BlockSpec((1, BN, EMB), lambda i: (0, i, 0))
    zispec = pl.BlockSpec((1, BN, EMB), lambda i: (1, i, 0))
    wspec = pl.BlockSpec((EMB, EMB), lambda i: (0, 0))
    bias = pl.BlockSpec((1, EMB), lambda i: (0, 0))
    return pl.pallas_call(
        functools.partial(_layer_body, final=final),
        grid=(NP // BN,),
        in_specs=[zpspec, zispec, wspec, bias, wspec, bias,
                  wspec, bias, wspec, bias],
        out_specs=pl.BlockSpec((BN, EMB), lambda i: (i, 0)),
        out_shape=jax.ShapeDtypeStruct((NP, EMB), jnp.float32),
    )(z2, z2, *weights)


AGG_BN = 1024
AGG_NBLK = NP // AGG_BN


def _agg_body(h_ref, bcol_ref, brow_ref, s_out, mean_out, mx_out, mn_out,
              s_acc, c_acc, mx_acc, mn_acc):
    i = pl.program_id(0)

    @pl.when(i == 0)
    def _():
        s_acc[...] = jnp.zeros((NG, EMB), jnp.float32)
        c_acc[...] = jnp.zeros((NG, EMB), jnp.float32)
        mx_acc[...] = jnp.full((NG, EMB), -jnp.inf, jnp.float32)
        mn_acc[...] = jnp.full((NG, EMB), jnp.inf, jnp.float32)

    hb = h_ref[...]
    bcol = bcol_ref[...]
    brow = brow_ref[...]

    onehot_t = (lax.broadcasted_iota(jnp.int32, (NG, AGG_BN), 0)
                == brow).astype(jnp.float32)
    s_acc[...] += jnp.dot(onehot_t, hb, preferred_element_type=jnp.float32)
    c_acc[...] += jnp.broadcast_to(
        jnp.sum(onehot_t, axis=1, keepdims=True), (NG, EMB))

    g_lo = jnp.min(bcol)
    g_hi = jnp.max(bcol)

    def gbody(g, carry):
        m = bcol == g
        mxr = jnp.max(jnp.where(m, hb, -jnp.inf), axis=0, keepdims=True)
        mnr = jnp.min(jnp.where(m, hb, jnp.inf), axis=0, keepdims=True)
        sel = lax.broadcasted_iota(jnp.int32, (NG, 1), 0) == g
        mx_acc[...] = jnp.where(sel, jnp.maximum(mx_acc[...], mxr), mx_acc[...])
        mn_acc[...] = jnp.where(sel, jnp.minimum(mn_acc[...], mnr), mn_acc[...])
        return carry

    lax.fori_loop(g_lo, g_hi + 1, gbody, 0)

    @pl.when(i == AGG_NBLK - 1)
    def _():
        s = s_acc[...]
        s_out[...] = s
        mean_out[...] = s / jnp.maximum(c_acc[...], 1.0)
        mx_out[...] = mx_acc[...]
        mn_out[...] = mn_acc[...]


def _agg_call(h, batch_col, batch_row):
    ospec = pl.BlockSpec((NG, EMB), lambda i: (0, 0))
    osd = jax.ShapeDtypeStruct((NG, EMB), jnp.float32)
    return pl.pallas_call(
        _agg_body,
        grid=(AGG_NBLK,),
        in_specs=[pl.BlockSpec((AGG_BN, EMB), lambda i: (i, 0)),
                  pl.BlockSpec((AGG_BN, 1), lambda i: (i, 0)),
                  pl.BlockSpec((1, AGG_BN), lambda i: (0, i))],
        out_specs=[ospec, ospec, ospec, ospec],
        out_shape=[osd, osd, osd, osd],
        scratch_shapes=[pltpu.VMEM((NG, EMB), jnp.float32)] * 4,
    )(h, batch_col, batch_row)


def _pad_edges(ei):
    src = jnp.concatenate(
        [ei[0], jnp.zeros((EPAD - E,), jnp.int32)]).reshape(NS, 1, NCHUNKS, CHUNK)
    dst = jnp.concatenate(
        [ei[1], jnp.full((EPAD - E,), NP - 1, jnp.int32)]).reshape(NS, 1, NCHUNKS, CHUNK)
    return jnp.concatenate([src, dst], axis=1)


def kernel(x, edge_index_pos, edge_index_inv, batch, type_emb, out_emb,
           W1, b1, W2, b2):
    e_all = jnp.stack([_pad_edges(edge_index_pos),
                       _pad_edges(edge_index_inv)])

    h = _embed_call(x, type_emb, out_emb)
    for l in range(NUM_LAYER):
        z2 = _seg_call(h, e_all)
        if isinstance(z2, (list, tuple)):
            z2 = z2[0]
        weights = (W1[l, 0], b1[l, 0].reshape(1, EMB),
                   W2[l, 0], b2[l, 0].reshape(1, EMB),
                   W1[l, 1], b1[l, 1].reshape(1, EMB),
                   W2[l, 1], b2[l, 1].reshape(1, EMB))
        h = _layer_call(z2, weights, final=(l == NUM_LAYER - 1))

    bpad = jnp.concatenate([batch, jnp.full((NP - N,), NG, jnp.int32)])
    s, mean, mx, mn = _agg_call(h, bpad.reshape(NP, 1), bpad.reshape(1, NP))

    hg = jnp.concatenate([s, mean, mx, mn], axis=1)
    hg = hg.reshape(NG, EMB, 4)
    hg = jnp.transpose(hg, (0, 2, 1))
    batch_mask = jnp.ones((NG, 4), dtype=bool)
    return (hg, batch_mask)


# speedup vs baseline: 3.0766x; 1.0915x over previous
import functools

import jax
import jax.numpy as jnp
from jax import lax
from jax.experimental import pallas as pl
from jax.experimental.pallas import tpu as pltpu
from jax.experimental.pallas import tpu_sc as plsc

N = 10000
E = 160000
EMB = 128
NUM_LAYER = 3
NG = 64

NS = 16
NP = 10240
RPS = NP // NS
CHUNK = 128
NCHUNKS = 80
NPAIR = NCHUNKS // 2
EPS = NCHUNKS * CHUNK
EPAD = NS * EPS

_mesh = plsc.VectorSubcoreMesh(core_axis_name="c", subcore_axis_name="s")


def _seg_body(h_hbm, e_hbm, z_hbm, idxr, bufa, bufb, acc, gsa, gsb, isem):
    cid = lax.axis_index("c")
    sid = lax.axis_index("s")
    r0 = sid * RPS

    pltpu.sync_copy(h_hbm.at[pl.ds(r0, RPS)], acc.at[pl.ds(r0, RPS)])

    plsc.subcore_barrier()

    pltpu.sync_copy(e_hbm.at[cid, sid, 0], idxr.at[0])
    pltpu.async_copy(h_hbm.at[idxr.at[0, 0, 0]], bufa, gsa)

    def body(j, carry):
        s = lax.rem(j, 2)
        s1 = lax.rem(j + 1, 2)

        @pl.when(j + 1 < NPAIR)
        def _():
            pltpu.async_copy(e_hbm.at[cid, sid, j + 1], idxr.at[s1],
                             isem.at[s1])

        pltpu.make_async_copy(h_hbm.at[idxr.at[s, 0, 0]], bufa, gsa).wait()
        pltpu.async_copy(h_hbm.at[idxr.at[s, 1, 0]], bufb, gsb)
        pltpu.sync_copy(bufa, acc.at[idxr.at[s, 0, 1]], add=True)

        @pl.when(j + 1 < NPAIR)
        def _():
            pltpu.make_async_copy(e_hbm.at[cid, sid, j + 1], idxr.at[s1],
                                  isem.at[s1]).wait()
            pltpu.async_copy(h_hbm.at[idxr.at[s1, 0, 0]], bufa, gsa)

        pltpu.make_async_copy(h_hbm.at[idxr.at[s, 1, 0]], bufb, gsb).wait()
        pltpu.sync_copy(bufb, acc.at[idxr.at[s, 1, 1]], add=True)
        return carry

    lax.fori_loop(0, NPAIR, body, 0)

    plsc.subcore_barrier()
    pltpu.sync_copy(acc.at[pl.ds(r0, RPS)], z_hbm.at[cid, pl.ds(r0, RPS)])


_seg_call = pl.kernel(
    _seg_body,
    out_type=[jax.ShapeDtypeStruct((2, NP, EMB), jnp.float32)],
    mesh=_mesh,
    scratch_types=[
        pltpu.VMEM((2, 2, 2, CHUNK), jnp.int32),
        pltpu.VMEM((CHUNK, EMB), jnp.float32),
        pltpu.VMEM((CHUNK, EMB), jnp.float32),
        pltpu.VMEM_SHARED((NP, EMB), jnp.float32),
        pltpu.SemaphoreType.DMA,
        pltpu.SemaphoreType.DMA,
        pltpu.SemaphoreType.DMA((2,)),
    ],
)


def _embed_body(x_ref, te_ref, oe_ref, out_ref):
    xb = x_ref[...]
    x0 = xb[:, 0:1].astype(jnp.float32)
    x1 = xb[:, 1:2].astype(jnp.float32)
    t0 = te_ref[0:1, :]
    t1 = te_ref[1:2, :]
    o0 = oe_ref[0:1, :]
    o1 = oe_ref[1:2, :]
    h = t0 + (t1 - t0) * x0 + o0 + (o1 - o0) * x1
    out_ref[:N, :] = h
    out_ref[N:, :] = jnp.zeros((NP - N, EMB), jnp.float32)


def _embed_call(x, type_emb, out_emb):
    return pl.pallas_call(
        _embed_body,
        out_shape=jax.ShapeDtypeStruct((NP, EMB), jnp.float32),
    )(x, type_emb, out_emb)


def _layer_body(zp_ref, zi_ref, w1p, b1p, w2p, b2p, w1i, b1i, w2i, b2i,
                out_ref, *, final):
    zp = zp_ref[0]
    a = jnp.maximum(
        jnp.dot(zp, w1p[...], preferred_element_type=jnp.float32) + b1p[...], 0.0)
    hp = jnp.dot(a, w2p[...], preferred_element_type=jnp.float32) + b2p[...]
    zi = zi_ref[0]
    b = jnp.maximum(
        jnp.dot(zi, w1i[...], preferred_element_type=jnp.float32) + b1i[...], 0.0)
    hi = jnp.dot(b, w2i[...], preferred_element_type=jnp.float32) + b2i[...]
    o = hp + hi
    if not final:
        o = jnp.maximum(o, 0.0)
    out_ref[...] = o


def _layer_call(z2, weights, final):
    BN = 1024
    zpspec = pl.BlockSpec((1, BN, EMB), lambda i: (0, i, 0))
    zispec = pl.BlockSpec((1, BN, EMB), lambda i: (1, i, 0))
    wspec = pl.BlockSpec((EMB, EMB), lambda i: (0, 0))
    bias = pl.BlockSpec((1, EMB), lambda i: (0, 0))
    return pl.pallas_call(
        functools.partial(_layer_body, final=final),
        grid=(NP // BN,),
        in_specs=[zpspec, zispec, wspec, bias, wspec, bias,
                  wspec, bias, wspec, bias],
        out_specs=pl.BlockSpec((BN, EMB), lambda i: (i, 0)),
        out_shape=jax.ShapeDtypeStruct((NP, EMB), jnp.float32),
    )(z2, z2, *weights)


AGG_BN = 1024
AGG_NBLK = NP // AGG_BN


def _agg_body(h_ref, bcol_ref, brow_ref, s_out, mean_out, mx_out, mn_out,
              s_acc, c_acc, mx_acc, mn_acc):
    i = pl.program_id(0)

    @pl.when(i == 0)
    def _():
        s_acc[...] = jnp.zeros((NG, EMB), jnp.float32)
        c_acc[...] = jnp.zeros((NG, EMB), jnp.float32)
        mx_acc[...] = jnp.full((NG, EMB), -jnp.inf, jnp.float32)
        mn_acc[...] = jnp.full((NG, EMB), jnp.inf, jnp.float32)

    hb = h_ref[...]
    bcol = bcol_ref[...]
    brow = brow_ref[...]

    onehot_t = (lax.broadcasted_iota(jnp.int32, (NG, AGG_BN), 0)
                == brow).astype(jnp.float32)
    s_acc[...] += jnp.dot(onehot_t, hb, preferred_element_type=jnp.float32)
    c_acc[...] += jnp.broadcast_to(
        jnp.sum(onehot_t, axis=1, keepdims=True), (NG, EMB))

    g_lo = jnp.min(bcol)
    g_hi = jnp.max(bcol)

    def gbody(g, carry):
        m = bcol == g
        mxr = jnp.max(jnp.where(m, hb, -jnp.inf), axis=0, keepdims=True)
        mnr = jnp.min(jnp.where(m, hb, jnp.inf), axis=0, keepdims=True)
        sel = lax.broadcasted_iota(jnp.int32, (NG, 1), 0) == g
        mx_acc[...] = jnp.where(sel, jnp.maximum(mx_acc[...], mxr), mx_acc[...])
        mn_acc[...] = jnp.where(sel, jnp.minimum(mn_acc[...], mnr), mn_acc[...])
        return carry

    lax.fori_loop(g_lo, g_hi + 1, gbody, 0)

    @pl.when(i == AGG_NBLK - 1)
    def _():
        s = s_acc[...]
        s_out[...] = s
        mean_out[...] = s / jnp.maximum(c_acc[...], 1.0)
        mx_out[...] = mx_acc[...]
        mn_out[...] = mn_acc[...]


def _agg_call(h, batch_col, batch_row):
    ospec = pl.BlockSpec((NG, EMB), lambda i: (0, 0))
    osd = jax.ShapeDtypeStruct((NG, EMB), jnp.float32)
    return pl.pallas_call(
        _agg_body,
        grid=(AGG_NBLK,),
        in_specs=[pl.BlockSpec((AGG_BN, EMB), lambda i: (i, 0)),
                  pl.BlockSpec((AGG_BN, 1), lambda i: (i, 0)),
                  pl.BlockSpec((1, AGG_BN), lambda i: (0, i))],
        out_specs=[ospec, ospec, ospec, ospec],
        out_shape=[osd, osd, osd, osd],
        scratch_shapes=[pltpu.VMEM((NG, EMB), jnp.float32)] * 4,
    )(h, batch_col, batch_row)


def _pad_edges(ei):
    src = jnp.concatenate(
        [ei[0], jnp.zeros((EPAD - E,), jnp.int32)]).reshape(NS, NPAIR, 2, 1, CHUNK)
    dst = jnp.concatenate(
        [ei[1], jnp.full((EPAD - E,), NP - 1, jnp.int32)]).reshape(NS, NPAIR, 2, 1, CHUNK)
    return jnp.concatenate([src, dst], axis=3)


def kernel(x, edge_index_pos, edge_index_inv, batch, type_emb, out_emb,
           W1, b1, W2, b2):
    e_all = jnp.stack([_pad_edges(edge_index_pos),
                       _pad_edges(edge_index_inv)])

    h = _embed_call(x, type_emb, out_emb)
    for l in range(NUM_LAYER):
        z2 = _seg_call(h, e_all)
        if isinstance(z2, (list, tuple)):
            z2 = z2[0]
        weights = (W1[l, 0], b1[l, 0].reshape(1, EMB),
                   W2[l, 0], b2[l, 0].reshape(1, EMB),
                   W1[l, 1], b1[l, 1].reshape(1, EMB),
                   W2[l, 1], b2[l, 1].reshape(1, EMB))
        h = _layer_call(z2, weights, final=(l == NUM_LAYER - 1))

    bpad = jnp.concatenate([batch, jnp.full((NP - N,), NG, jnp.int32)])
    s, mean, mx, mn = _agg_call(h, bpad.reshape(NP, 1), bpad.reshape(1, NP))

    hg = jnp.concatenate([s, mean, mx, mn], axis=1)
    hg = hg.reshape(NG, EMB, 4)
    hg = jnp.transpose(hg, (0, 2, 1))
    batch_mask = jnp.ones((NG, 4), dtype=bool)
    return (hg, batch_mask)


# speedup vs baseline: 3.1163x; 1.0129x over previous
import functools

import jax
import jax.numpy as jnp
from jax import lax
from jax.experimental import pallas as pl
from jax.experimental.pallas import tpu as pltpu
from jax.experimental.pallas import tpu_sc as plsc

N = 10000
E = 160000
EMB = 128
NUM_LAYER = 3
NG = 64

NS = 16
NP = 10240
RPS = NP // NS
CHUNK = 64
NCHUNKS = 160
NBUF = 5
NIDX = 7
EPS = NCHUNKS * CHUNK
EPAD = NS * EPS

_mesh = plsc.VectorSubcoreMesh(core_axis_name="c", subcore_axis_name="s")


def _seg_body(h_hbm, e_hbm, z_hbm, idxr, bufr, acc, gsem, ssem, isem):
    cid = lax.axis_index("c")
    sid = lax.axis_index("s")
    r0 = sid * RPS

    pltpu.sync_copy(h_hbm.at[pl.ds(r0, RPS)], acc.at[pl.ds(r0, RPS)])

    plsc.subcore_barrier()

    def fire_idx(x, carry):
        pltpu.async_copy(e_hbm.at[cid, sid, x], idxr.at[lax.rem(x, NIDX)],
                         isem.at[lax.rem(x, NIDX)])
        return carry

    def fire_gather(c, carry):
        xi = lax.rem(c, NIDX)
        b = lax.rem(c, NBUF)
        pltpu.make_async_copy(e_hbm.at[cid, sid, c], idxr.at[xi],
                              isem.at[xi]).wait()
        pltpu.async_copy(h_hbm.at[idxr.at[xi, 0]], bufr.at[b], gsem.at[b])
        return carry

    lax.fori_loop(0, 5, fire_idx, 0)
    lax.fori_loop(0, 3, fire_gather, 0)

    def body(ch, carry):
        b = lax.rem(ch, NBUF)
        xc = lax.rem(ch, NIDX)
        pltpu.make_async_copy(h_hbm.at[idxr.at[xc, 0]], bufr.at[b],
                              gsem.at[b]).wait()
        pltpu.async_copy(bufr.at[b], acc.at[idxr.at[xc, 1]], ssem.at[b],
                         add=True)
        f = ch + 3

        @pl.when(f < NCHUNKS)
        def _():
            bf = lax.rem(f, NBUF)
            xf = lax.rem(f, NIDX)

            @pl.when(f >= NBUF)
            def _():
                xo = lax.rem(f - NBUF, NIDX)
                pltpu.make_async_copy(bufr.at[bf], acc.at[idxr.at[xo, 1]],
                                      ssem.at[bf]).wait()

            pltpu.make_async_copy(e_hbm.at[cid, sid, f], idxr.at[xf],
                                  isem.at[xf]).wait()
            pltpu.async_copy(h_hbm.at[idxr.at[xf, 0]], bufr.at[bf],
                             gsem.at[bf])

        @pl.when(ch + 5 < NCHUNKS)
        def _():
            x = ch + 5
            pltpu.async_copy(e_hbm.at[cid, sid, x], idxr.at[lax.rem(x, NIDX)],
                             isem.at[lax.rem(x, NIDX)])

        return carry

    lax.fori_loop(0, NCHUNKS, body, 0)

    def drain(ch, carry):
        b = lax.rem(ch, NBUF)
        xc = lax.rem(ch, NIDX)
        pltpu.make_async_copy(bufr.at[b], acc.at[idxr.at[xc, 1]],
                              ssem.at[b]).wait()
        return carry

    lax.fori_loop(NCHUNKS - NBUF, NCHUNKS, drain, 0)

    plsc.subcore_barrier()
    pltpu.sync_copy(acc.at[pl.ds(r0, RPS)], z_hbm.at[cid, pl.ds(r0, RPS)])


_seg_call = pl.kernel(
    _seg_body,
    out_type=[jax.ShapeDtypeStruct((2, NP, EMB), jnp.float32)],
    mesh=_mesh,
    scratch_types=[
        pltpu.VMEM((NIDX, 2, CHUNK), jnp.int32),
        pltpu.VMEM((NBUF, CHUNK, EMB), jnp.float32),
        pltpu.VMEM_SHARED((NP, EMB), jnp.float32),
        pltpu.SemaphoreType.DMA((NBUF,)),
        pltpu.SemaphoreType.DMA((NBUF,)),
        pltpu.SemaphoreType.DMA((NIDX,)),
    ],
)


def _embed_body(x_ref, te_ref, oe_ref, out_ref):
    xb = x_ref[...]
    x0 = xb[:, 0:1].astype(jnp.float32)
    x1 = xb[:, 1:2].astype(jnp.float32)
    t0 = te_ref[0:1, :]
    t1 = te_ref[1:2, :]
    o0 = oe_ref[0:1, :]
    o1 = oe_ref[1:2, :]
    h = t0 + (t1 - t0) * x0 + o0 + (o1 - o0) * x1
    out_ref[:N, :] = h
    out_ref[N:, :] = jnp.zeros((NP - N, EMB), jnp.float32)


def _embed_call(x, type_emb, out_emb):
    return pl.pallas_call(
        _embed_body,
        out_shape=jax.ShapeDtypeStruct((NP, EMB), jnp.float32),
    )(x, type_emb, out_emb)


def _layer_body(zp_ref, zi_ref, w1p, b1p, w2p, b2p, w1i, b1i, w2i, b2i,
                out_ref, *, final):
    zp = zp_ref[0]
    a = jnp.maximum(
        jnp.dot(zp, w1p[...], preferred_element_type=jnp.float32) + b1p[...], 0.0)
    hp = jnp.dot(a, w2p[...], preferred_element_type=jnp.float32) + b2p[...]
    zi = zi_ref[0]
    b = jnp.maximum(
        jnp.dot(zi, w1i[...], preferred_element_type=jnp.float32) + b1i[...], 0.0)
    hi = jnp.dot(b, w2i[...], preferred_element_type=jnp.float32) + b2i[...]
    o = hp + hi
    if not final:
        o = jnp.maximum(o, 0.0)
    out_ref[...] = o


def _layer_call(z2, weights, final):
    BN = 1024
    zpspec = pl.BlockSpec((1, BN, EMB), lambda i: (0, i, 0))
    zispec = pl.BlockSpec((1, BN, EMB), lambda i: (1, i, 0))
    wspec = pl.BlockSpec((EMB, EMB), lambda i: (0, 0))
    bias = pl.BlockSpec((1, EMB), lambda i: (0, 0))
    return pl.pallas_call(
        functools.partial(_layer_body, final=final),
        grid=(NP // BN,),
        in_specs=[zpspec, zispec, wspec, bias, wspec, bias,
                  wspec, bias, wspec, bias],
        out_specs=pl.BlockSpec((BN, EMB), lambda i: (i, 0)),
        out_shape=jax.ShapeDtypeStruct((NP, EMB), jnp.float32),
    )(z2, z2, *weights)


AGG_BN = 1024
AGG_NBLK = NP // AGG_BN


def _agg_body(h_ref, bcol_ref, brow_ref, s_out, mean_out, mx_out, mn_out,
              s_acc, c_acc, mx_acc, mn_acc):
    i = pl.program_id(0)

    @pl.when(i == 0)
    def _():
        s_acc[...] = jnp.zeros((NG, EMB), jnp.float32)
        c_acc[...] = jnp.zeros((NG, EMB), jnp.float32)
        mx_acc[...] = jnp.full((NG, EMB), -jnp.inf, jnp.float32)
        mn_acc[...] = jnp.full((NG, EMB), jnp.inf, jnp.float32)

    hb = h_ref[...]
    bcol = bcol_ref[...]
    brow = brow_ref[...]

    onehot_t = (lax.broadcasted_iota(jnp.int32, (NG, AGG_BN), 0)
                == brow).astype(jnp.float32)
    s_acc[...] += jnp.dot(onehot_t, hb, preferred_element_type=jnp.float32)
    c_acc[...] += jnp.broadcast_to(
        jnp.sum(onehot_t, axis=1, keepdims=True), (NG, EMB))

    g_lo = jnp.min(bcol)
    g_hi = jnp.max(bcol)

    def gbody(g, carry):
        m = bcol == g
        mxr = jnp.max(jnp.where(m, hb, -jnp.inf), axis=0, keepdims=True)
        mnr = jnp.min(jnp.where(m, hb, jnp.inf), axis=0, keepdims=True)
        sel = lax.broadcasted_iota(jnp.int32, (NG, 1), 0) == g
        mx_acc[...] = jnp.where(sel, jnp.maximum(mx_acc[...], mxr), mx_acc[...])
        mn_acc[...] = jnp.where(sel, jnp.minimum(mn_acc[...], mnr), mn_acc[...])
        return carry

    lax.fori_loop(g_lo, g_hi + 1, gbody, 0)

    @pl.when(i == AGG_NBLK - 1)
    def _():
        s = s_acc[...]
        s_out[...] = s
        mean_out[...] = s / jnp.maximum(c_acc[...], 1.0)
        mx_out[...] = mx_acc[...]
        mn_out[...] = mn_acc[...]


def _agg_call(h, batch_col, batch_row):
    ospec = pl.BlockSpec((NG, EMB), lambda i: (0, 0))
    osd = jax.ShapeDtypeStruct((NG, EMB), jnp.float32)
    return pl.pallas_call(
        _agg_body,
        grid=(AGG_NBLK,),
        in_specs=[pl.BlockSpec((AGG_BN, EMB), lambda i: (i, 0)),
                  pl.BlockSpec((AGG_BN, 1), lambda i: (i, 0)),
                  pl.BlockSpec((1, AGG_BN), lambda i: (0, i))],
        out_specs=[ospec, ospec, ospec, ospec],
        out_shape=[osd, osd, osd, osd],
        scratch_shapes=[pltpu.VMEM((NG, EMB), jnp.float32)] * 4,
    )(h, batch_col, batch_row)


def _pad_edges(ei):
    src = jnp.concatenate(
        [ei[0], jnp.zeros((EPAD - E,), jnp.int32)]).reshape(NS, NCHUNKS, 1, CHUNK)
    dst = jnp.concatenate(
        [ei[1], jnp.full((EPAD - E,), NP - 1, jnp.int32)]).reshape(NS, NCHUNKS, 1, CHUNK)
    return jnp.concatenate([src, dst], axis=2)


def kernel(x, edge_index_pos, edge_index_inv, batch, type_emb, out_emb,
           W1, b1, W2, b2):
    e_all = jnp.stack([_pad_edges(edge_index_pos),
                       _pad_edges(edge_index_inv)])

    h = _embed_call(x, type_emb, out_emb)
    for l in range(NUM_LAYER):
        z2 = _seg_call(h, e_all)
        if isinstance(z2, (list, tuple)):
            z2 = z2[0]
        weights = (W1[l, 0], b1[l, 0].reshape(1, EMB),
                   W2[l, 0], b2[l, 0].reshape(1, EMB),
                   W1[l, 1], b1[l, 1].reshape(1, EMB),
                   W2[l, 1], b2[l, 1].reshape(1, EMB))
        h = _layer_call(z2, weights, final=(l == NUM_LAYER - 1))

    bpad = jnp.concatenate([batch, jnp.full((NP - N,), NG, jnp.int32)])
    s, mean, mx, mn = _agg_call(h, bpad.reshape(NP, 1), bpad.reshape(1, NP))

    hg = jnp.concatenate([s, mean, mx, mn], axis=1)
    hg = hg.reshape(NG, EMB, 4)
    hg = jnp.transpose(hg, (0, 2, 1))
    batch_mask = jnp.ones((NG, 4), dtype=bool)
    return (hg, batch_mask)


# speedup vs baseline: 8.4617x; 2.7153x over previous
import functools

import jax
import jax.numpy as jnp
from jax import lax
from jax.experimental import pallas as pl
from jax.experimental.pallas import tpu as pltpu
from jax.experimental.pallas import tpu_sc as plsc

N = 10000
E = 160000
EMB = 128
NUM_LAYER = 3
NG = 64

NS = 16
NP = 10240
RPS = NP // NS
CHUNK = 128
NCHUNKS = 80
GRP = 8
NGRP = NCHUNKS // GRP
EPS = NCHUNKS * CHUNK
EPAD = NS * EPS

_mesh = plsc.VectorSubcoreMesh(core_axis_name="c", subcore_axis_name="s")


def _seg_body(h_hbm, e_hbm, z_hbm, idxr, bufa, bufb, acc, gsa, gsb, isem):
    cid = lax.axis_index("c")
    sid = lax.axis_index("s")
    r0 = sid * RPS

    pltpu.sync_copy(h_hbm.at[pl.ds(r0, RPS)], acc.at[pl.ds(r0, RPS)])

    plsc.subcore_barrier()

    pltpu.sync_copy(e_hbm.at[cid, sid, 0], idxr.at[0])
    pltpu.async_copy(h_hbm.at[idxr.at[0, 0, 0]], bufa, gsa)

    def outer(g, carry):
        s = lax.rem(g, 2)
        s1 = lax.rem(g + 1, 2)

        @pl.when(g + 1 < NGRP)
        def _():
            pltpu.async_copy(e_hbm.at[cid, sid, g + 1], idxr.at[s1],
                             isem.at[s1])

        def inner(j, carry2):
            k0 = 2 * j
            k1 = 2 * j + 1
            pltpu.make_async_copy(h_hbm.at[idxr.at[s, 0, k0]], bufa,
                                  gsa).wait()
            pltpu.async_copy(h_hbm.at[idxr.at[s, 0, k1]], bufb, gsb)
            pltpu.sync_copy(bufa, acc.at[idxr.at[s, 1, k0]], add=True)

            @pl.when(k0 + 2 < GRP)
            def _():
                pltpu.async_copy(h_hbm.at[idxr.at[s, 0, k0 + 2]], bufa, gsa)

            pltpu.make_async_copy(h_hbm.at[idxr.at[s, 0, k1]], bufb,
                                  gsb).wait()
            pltpu.sync_copy(bufb, acc.at[idxr.at[s, 1, k1]], add=True)
            return carry2

        lax.fori_loop(0, GRP // 2, inner, 0)

        @pl.when(g + 1 < NGRP)
        def _():
            pltpu.make_async_copy(e_hbm.at[cid, sid, g + 1], idxr.at[s1],
                                  isem.at[s1]).wait()
            pltpu.async_copy(h_hbm.at[idxr.at[s1, 0, 0]], bufa, gsa)

        return carry

    lax.fori_loop(0, NGRP, outer, 0)

    plsc.subcore_barrier()
    pltpu.sync_copy(acc.at[pl.ds(r0, RPS)], z_hbm.at[cid, pl.ds(r0, RPS)])


_seg_call = pl.kernel(
    _seg_body,
    out_type=[jax.ShapeDtypeStruct((2, NP, EMB), jnp.float32)],
    mesh=_mesh,
    scratch_types=[
        pltpu.VMEM((2, 2, GRP, CHUNK), jnp.int32),
        pltpu.VMEM((CHUNK, EMB), jnp.float32),
        pltpu.VMEM((CHUNK, EMB), jnp.float32),
        pltpu.VMEM_SHARED((NP, EMB), jnp.float32),
        pltpu.SemaphoreType.DMA,
        pltpu.SemaphoreType.DMA,
        pltpu.SemaphoreType.DMA((2,)),
    ],
)


def _embed_body(x_ref, te_ref, oe_ref, out_ref):
    xb = x_ref[...]
    x0 = xb[:, 0:1].astype(jnp.float32)
    x1 = xb[:, 1:2].astype(jnp.float32)
    t0 = te_ref[0:1, :]
    t1 = te_ref[1:2, :]
    o0 = oe_ref[0:1, :]
    o1 = oe_ref[1:2, :]
    h = t0 + (t1 - t0) * x0 + o0 + (o1 - o0) * x1
    out_ref[:N, :] = h
    out_ref[N:, :] = jnp.zeros((NP - N, EMB), jnp.float32)


def _embed_call(x, type_emb, out_emb):
    return pl.pallas_call(
        _embed_body,
        out_shape=jax.ShapeDtypeStruct((NP, EMB), jnp.float32),
    )(x, type_emb, out_emb)


def _layer_body(zp_ref, zi_ref, w1p, b1p, w2p, b2p, w1i, b1i, w2i, b2i,
                out_ref, *, final):
    zp = zp_ref[0]
    a = jnp.maximum(
        jnp.dot(zp, w1p[...], preferred_element_type=jnp.float32) + b1p[...], 0.0)
    hp = jnp.dot(a, w2p[...], preferred_element_type=jnp.float32) + b2p[...]
    zi = zi_ref[0]
    b = jnp.maximum(
        jnp.dot(zi, w1i[...], preferred_element_type=jnp.float32) + b1i[...], 0.0)
    hi = jnp.dot(b, w2i[...], preferred_element_type=jnp.float32) + b2i[...]
    o = hp + hi
    if not final:
        o = jnp.maximum(o, 0.0)
    out_ref[...] = o


def _layer_call(z2, weights, final):
    BN = 1024
    zpspec = pl.BlockSpec((1, BN, EMB), lambda i: (0, i, 0))
    zispec = pl.BlockSpec((1, BN, EMB), lambda i: (1, i, 0))
    wspec = pl.BlockSpec((EMB, EMB), lambda i: (0, 0))
    bias = pl.BlockSpec((1, EMB), lambda i: (0, 0))
    return pl.pallas_call(
        functools.partial(_layer_body, final=final),
        grid=(NP // BN,),
        in_specs=[zpspec, zispec, wspec, bias, wspec, bias,
                  wspec, bias, wspec, bias],
        out_specs=pl.BlockSpec((BN, EMB), lambda i: (i, 0)),
        out_shape=jax.ShapeDtypeStruct((NP, EMB), jnp.float32),
    )(z2, z2, *weights)


AGG_BN = 1024
AGG_NBLK = NP // AGG_BN


def _agg_body(h_ref, bcol_ref, brow_ref, s_out, mean_out, mx_out, mn_out,
              s_acc, c_acc, mx_acc, mn_acc):
    i = pl.program_id(0)

    @pl.when(i == 0)
    def _():
        s_acc[...] = jnp.zeros((NG, EMB), jnp.float32)
        c_acc[...] = jnp.zeros((NG, EMB), jnp.float32)
        mx_acc[...] = jnp.full((NG, EMB), -jnp.inf, jnp.float32)
        mn_acc[...] = jnp.full((NG, EMB), jnp.inf, jnp.float32)

    hb = h_ref[...]
    bcol = bcol_ref[...]
    brow = brow_ref[...]

    onehot_t = (lax.broadcasted_iota(jnp.int32, (NG, AGG_BN), 0)
                == brow).astype(jnp.float32)
    s_acc[...] += jnp.dot(onehot_t, hb, preferred_element_type=jnp.float32)
    c_acc[...] += jnp.broadcast_to(
        jnp.sum(onehot_t, axis=1, keepdims=True), (NG, EMB))

    g_lo = jnp.min(bcol)
    g_hi = jnp.max(bcol)

    def gbody(g, carry):
        m = bcol == g
        mxr = jnp.max(jnp.where(m, hb, -jnp.inf), axis=0, keepdims=True)
        mnr = jnp.min(jnp.where(m, hb, jnp.inf), axis=0, keepdims=True)
        sel = lax.broadcasted_iota(jnp.int32, (NG, 1), 0) == g
        mx_acc[...] = jnp.where(sel, jnp.maximum(mx_acc[...], mxr), mx_acc[...])
        mn_acc[...] = jnp.where(sel, jnp.minimum(mn_acc[...], mnr), mn_acc[...])
        return carry

    lax.fori_loop(g_lo, g_hi + 1, gbody, 0)

    @pl.when(i == AGG_NBLK - 1)
    def _():
        s = s_acc[...]
        s_out[...] = s
        mean_out[...] = s / jnp.maximum(c_acc[...], 1.0)
        mx_out[...] = mx_acc[...]
        mn_out[...] = mn_acc[...]


def _agg_call(h, batch_col, batch_row):
    ospec = pl.BlockSpec((NG, EMB), lambda i: (0, 0))
    osd = jax.ShapeDtypeStruct((NG, EMB), jnp.float32)
    return pl.pallas_call(
        _agg_body,
        grid=(AGG_NBLK,),
        in_specs=[pl.BlockSpec((AGG_BN, EMB), lambda i: (i, 0)),
                  pl.BlockSpec((AGG_BN, 1), lambda i: (i, 0)),
                  pl.BlockSpec((1, AGG_BN), lambda i: (0, i))],
        out_specs=[ospec, ospec, ospec, ospec],
        out_shape=[osd, osd, osd, osd],
        scratch_shapes=[pltpu.VMEM((NG, EMB), jnp.float32)] * 4,
    )(h, batch_col, batch_row)


def _pad_edges(ei):
    npad = EPAD - E
    pad_src = jnp.arange(npad, dtype=jnp.int32) % N
    pad_dst = N + jnp.arange(npad, dtype=jnp.int32) % (NP - N)
    src = jnp.concatenate(
        [ei[0], pad_src]).reshape(NS, NGRP, 1, GRP, CHUNK)
    dst = jnp.concatenate(
        [ei[1], pad_dst]).reshape(NS, NGRP, 1, GRP, CHUNK)
    return jnp.concatenate([src, dst], axis=2)


def kernel(x, edge_index_pos, edge_index_inv, batch, type_emb, out_emb,
           W1, b1, W2, b2):
    e_all = jnp.stack([_pad_edges(edge_index_pos),
                       _pad_edges(edge_index_inv)])

    h = _embed_call(x, type_emb, out_emb)
    for l in range(NUM_LAYER):
        z2 = _seg_call(h, e_all)
        if isinstance(z2, (list, tuple)):
            z2 = z2[0]
        weights = (W1[l, 0], b1[l, 0].reshape(1, EMB),
                   W2[l, 0], b2[l, 0].reshape(1, EMB),
                   W1[l, 1], b1[l, 1].reshape(1, EMB),
                   W2[l, 1], b2[l, 1].reshape(1, EMB))
        h = _layer_call(z2, weights, final=(l == NUM_LAYER - 1))

    bpad = jnp.concatenate([batch, jnp.full((NP - N,), NG, jnp.int32)])
    s, mean, mx, mn = _agg_call(h, bpad.reshape(NP, 1), bpad.reshape(1, NP))

    hg = jnp.concatenate([s, mean, mx, mn], axis=1)
    hg = hg.reshape(NG, EMB, 4)
    hg = jnp.transpose(hg, (0, 2, 1))
    batch_mask = jnp.ones((NG, 4), dtype=bool)
    return (hg, batch_mask)


# speedup vs baseline: 9.3207x; 1.1015x over previous
import functools

import jax
import jax.numpy as jnp
from jax import lax
from jax.experimental import pallas as pl
from jax.experimental.pallas import tpu as pltpu
from jax.experimental.pallas import tpu_sc as plsc

N = 10000
E = 160000
EMB = 128
NUM_LAYER = 3
NG = 64

NS = 16
NP = 10240
RPS = NP // NS
CHUNK = 112
NCHUNKS = 96
NBUF = 3
GRP = 8
NGRP = NCHUNKS // GRP
EPS = NCHUNKS * CHUNK
EPAD = NS * EPS

_mesh = plsc.VectorSubcoreMesh(core_axis_name="c", subcore_axis_name="s")


def _seg_body(h_hbm, e_hbm, z_hbm, idxr, bufr, acc, gsem, ssem, isem):
    cid = lax.axis_index("c")
    sid = lax.axis_index("s")
    r0 = sid * RPS

    pltpu.sync_copy(h_hbm.at[pl.ds(r0, RPS)], acc.at[pl.ds(r0, RPS)])

    plsc.subcore_barrier()

    pltpu.sync_copy(e_hbm.at[cid, sid, 0], idxr.at[0])

    def fire_pro(c, carry):
        pltpu.async_copy(h_hbm.at[idxr.at[0, 0, c]], bufr.at[c], gsem.at[c])
        return carry

    lax.fori_loop(0, 2, fire_pro, 0)

    def body(ch, carry):
        b = lax.rem(ch, NBUF)
        g = lax.div(ch, GRP)
        s = lax.rem(g, 2)
        k = lax.rem(ch, GRP)
        pltpu.make_async_copy(h_hbm.at[idxr.at[s, 0, k]], bufr.at[b],
                              gsem.at[b]).wait()
        pltpu.async_copy(bufr.at[b], acc.at[idxr.at[s, 1, k]], ssem.at[b],
                         add=True)
        f = ch + 2

        @pl.when(f < NCHUNKS)
        def _():
            bf = lax.rem(f, NBUF)
            gf = lax.div(f, GRP)
            sf = lax.rem(gf, 2)
            kf = lax.rem(f, GRP)

            @pl.when(f >= NBUF)
            def _():
                fo = f - NBUF
                so = lax.rem(lax.div(fo, GRP), 2)
                ko = lax.rem(fo, GRP)
                pltpu.make_async_copy(bufr.at[bf], acc.at[idxr.at[so, 1, ko]],
                                      ssem.at[bf]).wait()

            @pl.when(kf == 0)
            def _():
                pltpu.make_async_copy(e_hbm.at[cid, sid, gf], idxr.at[sf],
                                      isem.at[sf]).wait()

            pltpu.async_copy(h_hbm.at[idxr.at[sf, 0, kf]], bufr.at[bf],
                             gsem.at[bf])

        @pl.when(jnp.logical_and(k == 0, ch + GRP < NCHUNKS))
        def _():
            pltpu.async_copy(e_hbm.at[cid, sid, g + 1],
                             idxr.at[lax.rem(g + 1, 2)],
                             isem.at[lax.rem(g + 1, 2)])

        return carry

    lax.fori_loop(0, NCHUNKS, body, 0)

    def drain(ch, carry):
        b = lax.rem(ch, NBUF)
        s = lax.rem(lax.div(ch, GRP), 2)
        k = lax.rem(ch, GRP)
        pltpu.make_async_copy(bufr.at[b], acc.at[idxr.at[s, 1, k]],
                              ssem.at[b]).wait()
        return carry

    lax.fori_loop(NCHUNKS - NBUF, NCHUNKS, drain, 0)

    plsc.subcore_barrier()
    pltpu.sync_copy(acc.at[pl.ds(r0, RPS)], z_hbm.at[cid, pl.ds(r0, RPS)])


_seg_call = pl.kernel(
    _seg_body,
    out_type=[jax.ShapeDtypeStruct((2, NP, EMB), jnp.float32)],
    mesh=_mesh,
    scratch_types=[
        pltpu.VMEM((2, 2, GRP, CHUNK), jnp.int32),
        pltpu.VMEM((NBUF, CHUNK, EMB), jnp.float32),
        pltpu.VMEM_SHARED((NP, EMB), jnp.float32),
        pltpu.SemaphoreType.DMA((NBUF,)),
        pltpu.SemaphoreType.DMA((NBUF,)),
        pltpu.SemaphoreType.DMA((2,)),
    ],
)


def _embed_body(x_ref, te_ref, oe_ref, out_ref):
    xb = x_ref[...]
    x0 = xb[:, 0:1].astype(jnp.float32)
    x1 = xb[:, 1:2].astype(jnp.float32)
    t0 = te_ref[0:1, :]
    t1 = te_ref[1:2, :]
    o0 = oe_ref[0:1, :]
    o1 = oe_ref[1:2, :]
    h = t0 + (t1 - t0) * x0 + o0 + (o1 - o0) * x1
    out_ref[:N, :] = h
    out_ref[N:, :] = jnp.zeros((NP - N, EMB), jnp.float32)


def _embed_call(x, type_emb, out_emb):
    return pl.pallas_call(
        _embed_body,
        out_shape=jax.ShapeDtypeStruct((NP, EMB), jnp.float32),
    )(x, type_emb, out_emb)


def _layer_body(zp_ref, zi_ref, w1p, b1p, w2p, b2p, w1i, b1i, w2i, b2i,
                out_ref, *, final):
    zp = zp_ref[0]
    a = jnp.maximum(
        jnp.dot(zp, w1p[...], preferred_element_type=jnp.float32) + b1p[...], 0.0)
    hp = jnp.dot(a, w2p[...], preferred_element_type=jnp.float32) + b2p[...]
    zi = zi_ref[0]
    b = jnp.maximum(
        jnp.dot(zi, w1i[...], preferred_element_type=jnp.float32) + b1i[...], 0.0)
    hi = jnp.dot(b, w2i[...], preferred_element_type=jnp.float32) + b2i[...]
    o = hp + hi
    if not final:
        o = jnp.maximum(o, 0.0)
    out_ref[...] = o


def _layer_call(z2, weights, final):
    BN = 1024
    zpspec = pl.BlockSpec((1, BN, EMB), lambda i: (0, i, 0))
    zispec = pl.BlockSpec((1, BN, EMB), lambda i: (1, i, 0))
    wspec = pl.BlockSpec((EMB, EMB), lambda i: (0, 0))
    bias = pl.BlockSpec((1, EMB), lambda i: (0, 0))
    return pl.pallas_call(
        functools.partial(_layer_body, final=final),
        grid=(NP // BN,),
        in_specs=[zpspec, zispec, wspec, bias, wspec, bias,
                  wspec, bias, wspec, bias],
        out_specs=pl.BlockSpec((BN, EMB), lambda i: (i, 0)),
        out_shape=jax.ShapeDtypeStruct((NP, EMB), jnp.float32),
    )(z2, z2, *weights)


AGG_BN = 1024
AGG_NBLK = NP // AGG_BN


def _agg_body(h_ref, bcol_ref, brow_ref, s_out, mean_out, mx_out, mn_out,
              s_acc, c_acc, mx_acc, mn_acc):
    i = pl.program_id(0)

    @pl.when(i == 0)
    def _():
        s_acc[...] = jnp.zeros((NG, EMB), jnp.float32)
        c_acc[...] = jnp.zeros((NG, EMB), jnp.float32)
        mx_acc[...] = jnp.full((NG, EMB), -jnp.inf, jnp.float32)
        mn_acc[...] = jnp.full((NG, EMB), jnp.inf, jnp.float32)

    hb = h_ref[...]
    bcol = bcol_ref[...]
    brow = brow_ref[...]

    onehot_t = (lax.broadcasted_iota(jnp.int32, (NG, AGG_BN), 0)
                == brow).astype(jnp.float32)
    s_acc[...] += jnp.dot(onehot_t, hb, preferred_element_type=jnp.float32)
    c_acc[...] += jnp.broadcast_to(
        jnp.sum(onehot_t, axis=1, keepdims=True), (NG, EMB))

    g_lo = jnp.min(bcol)
    g_hi = jnp.max(bcol)

    def gbody(g, carry):
        m = bcol == g
        mxr = jnp.max(jnp.where(m, hb, -jnp.inf), axis=0, keepdims=True)
        mnr = jnp.min(jnp.where(m, hb, jnp.inf), axis=0, keepdims=True)
        sel = lax.broadcasted_iota(jnp.int32, (NG, 1), 0) == g
        mx_acc[...] = jnp.where(sel, jnp.maximum(mx_acc[...], mxr), mx_acc[...])
        mn_acc[...] = jnp.where(sel, jnp.minimum(mn_acc[...], mnr), mn_acc[...])
        return carry

    lax.fori_loop(g_lo, g_hi + 1, gbody, 0)

    @pl.when(i == AGG_NBLK - 1)
    def _():
        s = s_acc[...]
        s_out[...] = s
        mean_out[...] = s / jnp.maximum(c_acc[...], 1.0)
        mx_out[...] = mx_acc[...]
        mn_out[...] = mn_acc[...]


def _agg_call(h, batch_col, batch_row):
    ospec = pl.BlockSpec((NG, EMB), lambda i: (0, 0))
    osd = jax.ShapeDtypeStruct((NG, EMB), jnp.float32)
    return pl.pallas_call(
        _agg_body,
        grid=(AGG_NBLK,),
        in_specs=[pl.BlockSpec((AGG_BN, EMB), lambda i: (i, 0)),
                  pl.BlockSpec((AGG_BN, 1), lambda i: (i, 0)),
                  pl.BlockSpec((1, AGG_BN), lambda i: (0, i))],
        out_specs=[ospec, ospec, ospec, ospec],
        out_shape=[osd, osd, osd, osd],
        scratch_shapes=[pltpu.VMEM((NG, EMB), jnp.float32)] * 4,
    )(h, batch_col, batch_row)


def _pad_edges(ei):
    npad = EPAD - E
    pad_src = jnp.arange(npad, dtype=jnp.int32) % N
    pad_dst = N + jnp.arange(npad, dtype=jnp.int32) % (NP - N)
    src = jnp.concatenate(
        [ei[0], pad_src]).reshape(NS, NGRP, 1, GRP, CHUNK)
    dst = jnp.concatenate(
        [ei[1], pad_dst]).reshape(NS, NGRP, 1, GRP, CHUNK)
    return jnp.concatenate([src, dst], axis=2)


def kernel(x, edge_index_pos, edge_index_inv, batch, type_emb, out_emb,
           W1, b1, W2, b2):
    e_all = jnp.stack([_pad_edges(edge_index_pos),
                       _pad_edges(edge_index_inv)])

    h = _embed_call(x, type_emb, out_emb)
    for l in range(NUM_LAYER):
        z2 = _seg_call(h, e_all)
        if isinstance(z2, (list, tuple)):
            z2 = z2[0]
        weights = (W1[l, 0], b1[l, 0].reshape(1, EMB),
                   W2[l, 0], b2[l, 0].reshape(1, EMB),
                   W1[l, 1], b1[l, 1].reshape(1, EMB),
                   W2[l, 1], b2[l, 1].reshape(1, EMB))
        h = _layer_call(z2, weights, final=(l == NUM_LAYER - 1))

    bpad = jnp.concatenate([batch, jnp.full((NP - N,), NG, jnp.int32)])
    s, mean, mx, mn = _agg_call(h, bpad.reshape(NP, 1), bpad.reshape(1, NP))

    hg = jnp.concatenate([s, mean, mx, mn], axis=1)
    hg = hg.reshape(NG, EMB, 4)
    hg = jnp.transpose(hg, (0, 2, 1))
    batch_mask = jnp.ones((NG, 4), dtype=bool)
    return (hg, batch_mask)


# speedup vs baseline: 9.5021x; 1.0195x over previous
import functools

import jax
import jax.numpy as jnp
from jax import lax
from jax.experimental import pallas as pl
from jax.experimental.pallas import tpu as pltpu
from jax.experimental.pallas import tpu_sc as plsc

N = 10000
E = 160000
EMB = 128
NUM_LAYER = 3
NG = 64

NS = 16
NP = 10240
RPS = NP // NS
CHUNK = 112
NCHUNKS = 96
NBUF = 3
GRP = 8
NGRP = NCHUNKS // GRP
EPS = NCHUNKS * CHUNK
EPAD = NS * EPS

_mesh = plsc.VectorSubcoreMesh(core_axis_name="c", subcore_axis_name="s")


def _seg_body(h_hbm, e_hbm, z_hbm, idxr, bufr, acc, gsem, ssem, isem):
    cid = lax.axis_index("c")
    sid = lax.axis_index("s")
    r0 = sid * RPS

    pltpu.sync_copy(h_hbm.at[pl.ds(r0, RPS)], acc.at[pl.ds(r0, RPS)])

    plsc.subcore_barrier()

    pltpu.sync_copy(e_hbm.at[cid, sid, 0], idxr.at[0])

    def fire_pro(c, carry):
        pltpu.async_copy(h_hbm.at[idxr.at[0, 0, c]], bufr.at[c], gsem.at[c])
        return carry

    lax.fori_loop(0, 2, fire_pro, 0)

    def body(ch, carry):
        b = lax.rem(ch, NBUF)
        g = lax.div(ch, GRP)
        s = lax.rem(g, 2)
        k = lax.rem(ch, GRP)
        pltpu.make_async_copy(h_hbm.at[idxr.at[s, 0, k]], bufr.at[b],
                              gsem.at[b]).wait()
        pltpu.async_copy(bufr.at[b], acc.at[idxr.at[s, 1, k]], ssem.at[b],
                         add=True)
        f = ch + 2

        @pl.when(f < NCHUNKS)
        def _():
            bf = lax.rem(f, NBUF)
            gf = lax.div(f, GRP)
            sf = lax.rem(gf, 2)
            kf = lax.rem(f, GRP)

            @pl.when(f >= NBUF)
            def _():
                fo = f - NBUF
                so = lax.rem(lax.div(fo, GRP), 2)
                ko = lax.rem(fo, GRP)
                pltpu.make_async_copy(bufr.at[bf], acc.at[idxr.at[so, 1, ko]],
                                      ssem.at[bf]).wait()

            @pl.when(kf == 0)
            def _():
                pltpu.make_async_copy(e_hbm.at[cid, sid, gf], idxr.at[sf],
                                      isem.at[sf]).wait()

            pltpu.async_copy(h_hbm.at[idxr.at[sf, 0, kf]], bufr.at[bf],
                             gsem.at[bf])

        @pl.when(jnp.logical_and(k == 0, ch + GRP < NCHUNKS))
        def _():
            pltpu.async_copy(e_hbm.at[cid, sid, g + 1],
                             idxr.at[lax.rem(g + 1, 2)],
                             isem.at[lax.rem(g + 1, 2)])

        return carry

    lax.fori_loop(0, NCHUNKS, body, 0)

    def drain(ch, carry):
        b = lax.rem(ch, NBUF)
        s = lax.rem(lax.div(ch, GRP), 2)
        k = lax.rem(ch, GRP)
        pltpu.make_async_copy(bufr.at[b], acc.at[idxr.at[s, 1, k]],
                              ssem.at[b]).wait()
        return carry

    lax.fori_loop(NCHUNKS - NBUF, NCHUNKS, drain, 0)

    plsc.subcore_barrier()
    pltpu.sync_copy(acc.at[pl.ds(r0, RPS)], z_hbm.at[cid, pl.ds(r0, RPS)])


_seg_call = pl.kernel(
    _seg_body,
    out_type=[jax.ShapeDtypeStruct((2, NP, EMB), jnp.float32)],
    mesh=_mesh,
    scratch_types=[
        pltpu.VMEM((2, 2, GRP, CHUNK), jnp.int32),
        pltpu.VMEM((NBUF, CHUNK, EMB), jnp.float32),
        pltpu.VMEM_SHARED((NP, EMB), jnp.float32),
        pltpu.SemaphoreType.DMA((NBUF,)),
        pltpu.SemaphoreType.DMA((NBUF,)),
        pltpu.SemaphoreType.DMA((2,)),
    ],
)


def _embed_body(x_ref, te_ref, oe_ref, out_ref):
    xb = x_ref[...]
    x0 = xb[:, 0:1].astype(jnp.float32)
    x1 = xb[:, 1:2].astype(jnp.float32)
    t0 = te_ref[0:1, :]
    t1 = te_ref[1:2, :]
    o0 = oe_ref[0:1, :]
    o1 = oe_ref[1:2, :]
    h = t0 + (t1 - t0) * x0 + o0 + (o1 - o0) * x1
    out_ref[:N, :] = h
    out_ref[N:, :] = jnp.zeros((NP - N, EMB), jnp.float32)


def _embed_call(x, type_emb, out_emb):
    return pl.pallas_call(
        _embed_body,
        out_shape=jax.ShapeDtypeStruct((NP, EMB), jnp.float32),
    )(x, type_emb, out_emb)


def _layer_body(zp_ref, zi_ref, w1p, b1p, w2p, b2p, w1i, b1i, w2i, b2i,
                out_ref, *, final):
    zp = zp_ref[0]
    a = jnp.maximum(
        jnp.dot(zp, w1p[...], preferred_element_type=jnp.float32) + b1p[...], 0.0)
    hp = jnp.dot(a, w2p[...], preferred_element_type=jnp.float32) + b2p[...]
    zi = zi_ref[0]
    b = jnp.maximum(
        jnp.dot(zi, w1i[...], preferred_element_type=jnp.float32) + b1i[...], 0.0)
    hi = jnp.dot(b, w2i[...], preferred_element_type=jnp.float32) + b2i[...]
    o = hp + hi
    if not final:
        o = jnp.maximum(o, 0.0)
    out_ref[...] = o


def _layer_call(z2, weights, final):
    BN = 1024
    zpspec = pl.BlockSpec((1, BN, EMB), lambda i: (0, i, 0))
    zispec = pl.BlockSpec((1, BN, EMB), lambda i: (1, i, 0))
    wspec = pl.BlockSpec((EMB, EMB), lambda i: (0, 0))
    bias = pl.BlockSpec((1, EMB), lambda i: (0, 0))
    return pl.pallas_call(
        functools.partial(_layer_body, final=final),
        grid=(NP // BN,),
        in_specs=[zpspec, zispec, wspec, bias, wspec, bias,
                  wspec, bias, wspec, bias],
        out_specs=pl.BlockSpec((BN, EMB), lambda i: (i, 0)),
        out_shape=jax.ShapeDtypeStruct((NP, EMB), jnp.float32),
    )(z2, z2, *weights)


AGG_BN = 1024
AGG_NBLK = NP // AGG_BN


def _final_body(zp_ref, zi_ref, w1p, b1p, w2p, b2p, w1i, b1i, w2i, b2i,
                bcol_ref, brow_ref, s_out, mean_out, mx_out, mn_out,
                s_acc, c_acc, mx_acc, mn_acc):
    i = pl.program_id(0)

    @pl.when(i == 0)
    def _():
        s_acc[...] = jnp.zeros((NG, EMB), jnp.float32)
        c_acc[...] = jnp.zeros((NG, EMB), jnp.float32)
        mx_acc[...] = jnp.full((NG, EMB), -jnp.inf, jnp.float32)
        mn_acc[...] = jnp.full((NG, EMB), jnp.inf, jnp.float32)

    zp = zp_ref[0]
    a = jnp.maximum(
        jnp.dot(zp, w1p[...], preferred_element_type=jnp.float32) + b1p[...], 0.0)
    hp = jnp.dot(a, w2p[...], preferred_element_type=jnp.float32) + b2p[...]
    zi = zi_ref[0]
    b = jnp.maximum(
        jnp.dot(zi, w1i[...], preferred_element_type=jnp.float32) + b1i[...], 0.0)
    hi = jnp.dot(b, w2i[...], preferred_element_type=jnp.float32) + b2i[...]
    hb = hp + hi

    bcol = bcol_ref[...]
    brow = brow_ref[...]

    onehot_t = (lax.broadcasted_iota(jnp.int32, (NG, AGG_BN), 0)
                == brow).astype(jnp.float32)
    s_acc[...] += jnp.dot(onehot_t, hb, preferred_element_type=jnp.float32)
    c_acc[...] += jnp.broadcast_to(
        jnp.sum(onehot_t, axis=1, keepdims=True), (NG, EMB))

    g_lo = jnp.min(bcol)
    g_hi = jnp.max(bcol)

    def gbody(g, carry):
        m = bcol == g
        mxr = jnp.max(jnp.where(m, hb, -jnp.inf), axis=0, keepdims=True)
        mnr = jnp.min(jnp.where(m, hb, jnp.inf), axis=0, keepdims=True)
        sel = lax.broadcasted_iota(jnp.int32, (NG, 1), 0) == g
        mx_acc[...] = jnp.where(sel, jnp.maximum(mx_acc[...], mxr), mx_acc[...])
        mn_acc[...] = jnp.where(sel, jnp.minimum(mn_acc[...], mnr), mn_acc[...])
        return carry

    lax.fori_loop(g_lo, g_hi + 1, gbody, 0)

    @pl.when(i == AGG_NBLK - 1)
    def _():
        s = s_acc[...]
        s_out[...] = s
        mean_out[...] = s / jnp.maximum(c_acc[...], 1.0)
        mx_out[...] = mx_acc[...]
        mn_out[...] = mn_acc[...]


def _final_call(z2, weights, batch_col, batch_row):
    zpspec = pl.BlockSpec((1, AGG_BN, EMB), lambda i: (0, i, 0))
    zispec = pl.BlockSpec((1, AGG_BN, EMB), lambda i: (1, i, 0))
    wspec = pl.BlockSpec((EMB, EMB), lambda i: (0, 0))
    bias = pl.BlockSpec((1, EMB), lambda i: (0, 0))
    ospec = pl.BlockSpec((NG, EMB), lambda i: (0, 0))
    osd = jax.ShapeDtypeStruct((NG, EMB), jnp.float32)
    return pl.pallas_call(
        _final_body,
        grid=(AGG_NBLK,),
        in_specs=[zpspec, zispec, wspec, bias, wspec, bias,
                  wspec, bias, wspec, bias,
                  pl.BlockSpec((AGG_BN, 1), lambda i: (i, 0)),
                  pl.BlockSpec((1, AGG_BN), lambda i: (0, i))],
        out_specs=[ospec, ospec, ospec, ospec],
        out_shape=[osd, osd, osd, osd],
        scratch_shapes=[pltpu.VMEM((NG, EMB), jnp.float32)] * 4,
    )(z2, z2, *weights, batch_col, batch_row)


def _pad_edges(ei):
    npad = EPAD - E
    pad_src = jnp.arange(npad, dtype=jnp.int32) % N
    pad_dst = N + jnp.arange(npad, dtype=jnp.int32) % (NP - N)
    src = jnp.concatenate(
        [ei[0], pad_src]).reshape(NS, NGRP, 1, GRP, CHUNK)
    dst = jnp.concatenate(
        [ei[1], pad_dst]).reshape(NS, NGRP, 1, GRP, CHUNK)
    return jnp.concatenate([src, dst], axis=2)


def kernel(x, edge_index_pos, edge_index_inv, batch, type_emb, out_emb,
           W1, b1, W2, b2):
    e_all = jnp.stack([_pad_edges(edge_index_pos),
                       _pad_edges(edge_index_inv)])

    bpad = jnp.concatenate([batch, jnp.full((NP - N,), NG, jnp.int32)])

    h = _embed_call(x, type_emb, out_emb)
    for l in range(NUM_LAYER):
        z2 = _seg_call(h, e_all)
        if isinstance(z2, (list, tuple)):
            z2 = z2[0]
        weights = (W1[l, 0], b1[l, 0].reshape(1, EMB),
                   W2[l, 0], b2[l, 0].reshape(1, EMB),
                   W1[l, 1], b1[l, 1].reshape(1, EMB),
                   W2[l, 1], b2[l, 1].reshape(1, EMB))
        if l < NUM_LAYER - 1:
            h = _layer_call(z2, weights, final=False)
        else:
            s, mean, mx, mn = _final_call(
                z2, weights, bpad.reshape(NP, 1), bpad.reshape(1, NP))

    hg = jnp.concatenate([s, mean, mx, mn], axis=1)
    hg = hg.reshape(NG, EMB, 4)
    hg = jnp.transpose(hg, (0, 2, 1))
    batch_mask = jnp.ones((NG, 4), dtype=bool)
    return (hg, batch_mask)


# speedup vs baseline: 9.6922x; 1.0200x over previous
import functools

import jax
import jax.numpy as jnp
from jax import lax
from jax.experimental import pallas as pl
from jax.experimental.pallas import tpu as pltpu
from jax.experimental.pallas import tpu_sc as plsc

N = 10000
E = 160000
EMB = 128
NUM_LAYER = 3
NG = 64

NS = 16
NP = 10240
RPS = NP // NS
CHUNK = 112
NCHUNKS = 92
NBUF = 3
GRP = 4
NGRP = NCHUNKS // GRP
EPS = NCHUNKS * CHUNK
EPAD = NS * EPS

_mesh = plsc.VectorSubcoreMesh(core_axis_name="c", subcore_axis_name="s")


def _seg_body(h_hbm, e_hbm, z_hbm, idxr, bufr, acc, gsem, ssem, isem):
    cid = lax.axis_index("c")
    sid = lax.axis_index("s")
    r0 = sid * RPS

    pltpu.sync_copy(h_hbm.at[pl.ds(r0, RPS)], acc.at[pl.ds(r0, RPS)])

    plsc.subcore_barrier()

    pltpu.sync_copy(e_hbm.at[cid, sid, 0], idxr.at[0])

    def fire_pro(c, carry):
        pltpu.async_copy(h_hbm.at[idxr.at[0, 0, c]], bufr.at[c], gsem.at[c])
        return carry

    lax.fori_loop(0, 2, fire_pro, 0)

    def body(ch, carry):
        b = lax.rem(ch, NBUF)
        g = lax.div(ch, GRP)
        s = lax.rem(g, 2)
        k = lax.rem(ch, GRP)
        pltpu.make_async_copy(h_hbm.at[idxr.at[s, 0, k]], bufr.at[b],
                              gsem.at[b]).wait()
        pltpu.async_copy(bufr.at[b], acc.at[idxr.at[s, 1, k]], ssem.at[b],
                         add=True)
        f = ch + 2

        @pl.when(f < NCHUNKS)
        def _():
            bf = lax.rem(f, NBUF)
            gf = lax.div(f, GRP)
            sf = lax.rem(gf, 2)
            kf = lax.rem(f, GRP)

            @pl.when(f >= NBUF)
            def _():
                fo = f - NBUF
                so = lax.rem(lax.div(fo, GRP), 2)
                ko = lax.rem(fo, GRP)
                pltpu.make_async_copy(bufr.at[bf], acc.at[idxr.at[so, 1, ko]],
                                      ssem.at[bf]).wait()

            @pl.when(kf == 0)
            def _():
                pltpu.make_async_copy(e_hbm.at[cid, sid, gf], idxr.at[sf],
                                      isem.at[sf]).wait()

            pltpu.async_copy(h_hbm.at[idxr.at[sf, 0, kf]], bufr.at[bf],
                             gsem.at[bf])

        @pl.when(jnp.logical_and(k == 0, ch + GRP < NCHUNKS))
        def _():
            pltpu.async_copy(e_hbm.at[cid, sid, g + 1],
                             idxr.at[lax.rem(g + 1, 2)],
                             isem.at[lax.rem(g + 1, 2)])

        return carry

    lax.fori_loop(0, NCHUNKS, body, 0)

    def drain(ch, carry):
        b = lax.rem(ch, NBUF)
        s = lax.rem(lax.div(ch, GRP), 2)
        k = lax.rem(ch, GRP)
        pltpu.make_async_copy(bufr.at[b], acc.at[idxr.at[s, 1, k]],
                              ssem.at[b]).wait()
        return carry

    lax.fori_loop(NCHUNKS - NBUF, NCHUNKS, drain, 0)

    plsc.subcore_barrier()
    pltpu.sync_copy(acc.at[pl.ds(r0, RPS)], z_hbm.at[cid, pl.ds(r0, RPS)])


_seg_call = pl.kernel(
    _seg_body,
    out_type=[jax.ShapeDtypeStruct((2, NP, EMB), jnp.float32)],
    mesh=_mesh,
    scratch_types=[
        pltpu.VMEM((2, 2, GRP, CHUNK), jnp.int32),
        pltpu.VMEM((NBUF, CHUNK, EMB), jnp.float32),
        pltpu.VMEM_SHARED((NP, EMB), jnp.float32),
        pltpu.SemaphoreType.DMA((NBUF,)),
        pltpu.SemaphoreType.DMA((NBUF,)),
        pltpu.SemaphoreType.DMA((2,)),
    ],
)


def _embed_body(x_ref, te_ref, oe_ref, out_ref):
    xb = x_ref[...]
    x0 = xb[:, 0:1].astype(jnp.float32)
    x1 = xb[:, 1:2].astype(jnp.float32)
    t0 = te_ref[0:1, :]
    t1 = te_ref[1:2, :]
    o0 = oe_ref[0:1, :]
    o1 = oe_ref[1:2, :]
    h = t0 + (t1 - t0) * x0 + o0 + (o1 - o0) * x1
    out_ref[:N, :] = h
    out_ref[N:, :] = jnp.zeros((NP - N, EMB), jnp.float32)


def _embed_call(x, type_emb, out_emb):
    return pl.pallas_call(
        _embed_body,
        out_shape=jax.ShapeDtypeStruct((NP, EMB), jnp.float32),
    )(x, type_emb, out_emb)


def _layer_body(zp_ref, zi_ref, w1p, b1p, w2p, b2p, w1i, b1i, w2i, b2i,
                out_ref, *, final):
    zp = zp_ref[0]
    a = jnp.maximum(
        jnp.dot(zp, w1p[...], preferred_element_type=jnp.float32) + b1p[...], 0.0)
    hp = jnp.dot(a, w2p[...], preferred_element_type=jnp.float32) + b2p[...]
    zi = zi_ref[0]
    b = jnp.maximum(
        jnp.dot(zi, w1i[...], preferred_element_type=jnp.float32) + b1i[...], 0.0)
    hi = jnp.dot(b, w2i[...], preferred_element_type=jnp.float32) + b2i[...]
    o = hp + hi
    if not final:
        o = jnp.maximum(o, 0.0)
    out_ref[...] = o


def _layer_call(z2, weights, final):
    BN = 1024
    zpspec = pl.BlockSpec((1, BN, EMB), lambda i: (0, i, 0))
    zispec = pl.BlockSpec((1, BN, EMB), lambda i: (1, i, 0))
    wspec = pl.BlockSpec((EMB, EMB), lambda i: (0, 0))
    bias = pl.BlockSpec((1, EMB), lambda i: (0, 0))
    return pl.pallas_call(
        functools.partial(_layer_body, final=final),
        grid=(NP // BN,),
        in_specs=[zpspec, zispec, wspec, bias, wspec, bias,
                  wspec, bias, wspec, bias],
        out_specs=pl.BlockSpec((BN, EMB), lambda i: (i, 0)),
        out_shape=jax.ShapeDtypeStruct((NP, EMB), jnp.float32),
    )(z2, z2, *weights)


AGG_BN = 1024
AGG_NBLK = NP // AGG_BN


def _final_body(zp_ref, zi_ref, w1p, b1p, w2p, b2p, w1i, b1i, w2i, b2i,
                bcol_ref, brow_ref, s_out, mean_out, mx_out, mn_out,
                s_acc, c_acc, mx_acc, mn_acc):
    i = pl.program_id(0)

    @pl.when(i == 0)
    def _():
        s_acc[...] = jnp.zeros((NG, EMB), jnp.float32)
        c_acc[...] = jnp.zeros((NG, EMB), jnp.float32)
        mx_acc[...] = jnp.full((NG, EMB), -jnp.inf, jnp.float32)
        mn_acc[...] = jnp.full((NG, EMB), jnp.inf, jnp.float32)

    zp = zp_ref[0]
    a = jnp.maximum(
        jnp.dot(zp, w1p[...], preferred_element_type=jnp.float32) + b1p[...], 0.0)
    hp = jnp.dot(a, w2p[...], preferred_element_type=jnp.float32) + b2p[...]
    zi = zi_ref[0]
    b = jnp.maximum(
        jnp.dot(zi, w1i[...], preferred_element_type=jnp.float32) + b1i[...], 0.0)
    hi = jnp.dot(b, w2i[...], preferred_element_type=jnp.float32) + b2i[...]
    hb = hp + hi

    bcol = bcol_ref[...]
    brow = brow_ref[...]

    onehot_t = (lax.broadcasted_iota(jnp.int32, (NG, AGG_BN), 0)
                == brow).astype(jnp.float32)
    s_acc[...] += jnp.dot(onehot_t, hb, preferred_element_type=jnp.float32)
    c_acc[...] += jnp.broadcast_to(
        jnp.sum(onehot_t, axis=1, keepdims=True), (NG, EMB))

    g_lo = jnp.min(bcol)
    g_hi = jnp.max(bcol)

    def gbody(g, carry):
        m = bcol == g
        mxr = jnp.max(jnp.where(m, hb, -jnp.inf), axis=0, keepdims=True)
        mnr = jnp.min(jnp.where(m, hb, jnp.inf), axis=0, keepdims=True)
        sel = lax.broadcasted_iota(jnp.int32, (NG, 1), 0) == g
        mx_acc[...] = jnp.where(sel, jnp.maximum(mx_acc[...], mxr), mx_acc[...])
        mn_acc[...] = jnp.where(sel, jnp.minimum(mn_acc[...], mnr), mn_acc[...])
        return carry

    lax.fori_loop(g_lo, g_hi + 1, gbody, 0)

    @pl.when(i == AGG_NBLK - 1)
    def _():
        s = s_acc[...]
        s_out[...] = s
        mean_out[...] = s / jnp.maximum(c_acc[...], 1.0)
        mx_out[...] = mx_acc[...]
        mn_out[...] = mn_acc[...]


def _final_call(z2, weights, batch_col, batch_row):
    zpspec = pl.BlockSpec((1, AGG_BN, EMB), lambda i: (0, i, 0))
    zispec = pl.BlockSpec((1, AGG_BN, EMB), lambda i: (1, i, 0))
    wspec = pl.BlockSpec((EMB, EMB), lambda i: (0, 0))
    bias = pl.BlockSpec((1, EMB), lambda i: (0, 0))
    ospec = pl.BlockSpec((NG, EMB), lambda i: (0, 0))
    osd = jax.ShapeDtypeStruct((NG, EMB), jnp.float32)
    return pl.pallas_call(
        _final_body,
        grid=(AGG_NBLK,),
        in_specs=[zpspec, zispec, wspec, bias, wspec, bias,
                  wspec, bias, wspec, bias,
                  pl.BlockSpec((AGG_BN, 1), lambda i: (i, 0)),
                  pl.BlockSpec((1, AGG_BN), lambda i: (0, i))],
        out_specs=[ospec, ospec, ospec, ospec],
        out_shape=[osd, osd, osd, osd],
        scratch_shapes=[pltpu.VMEM((NG, EMB), jnp.float32)] * 4,
    )(z2, z2, *weights, batch_col, batch_row)


def _pad_edges(ei):
    npad = EPAD - E
    pad_src = jnp.arange(npad, dtype=jnp.int32) % N
    pad_dst = N + jnp.arange(npad, dtype=jnp.int32) % (NP - N)
    src = jnp.concatenate(
        [ei[0], pad_src]).reshape(NS, NGRP, 1, GRP, CHUNK)
    dst = jnp.concatenate(
        [ei[1], pad_dst]).reshape(NS, NGRP, 1, GRP, CHUNK)
    return jnp.concatenate([src, dst], axis=2)


def kernel(x, edge_index_pos, edge_index_inv, batch, type_emb, out_emb,
           W1, b1, W2, b2):
    e_all = jnp.stack([_pad_edges(edge_index_pos),
                       _pad_edges(edge_index_inv)])

    bpad = jnp.concatenate([batch, jnp.full((NP - N,), NG, jnp.int32)])

    h = _embed_call(x, type_emb, out_emb)
    for l in range(NUM_LAYER):
        z2 = _seg_call(h, e_all)
        if isinstance(z2, (list, tuple)):
            z2 = z2[0]
        weights = (W1[l, 0], b1[l, 0].reshape(1, EMB),
                   W2[l, 0], b2[l, 0].reshape(1, EMB),
                   W1[l, 1], b1[l, 1].reshape(1, EMB),
                   W2[l, 1], b2[l, 1].reshape(1, EMB))
        if l < NUM_LAYER - 1:
            h = _layer_call(z2, weights, final=False)
        else:
            s, mean, mx, mn = _final_call(
                z2, weights, bpad.reshape(NP, 1), bpad.reshape(1, NP))

    hg = jnp.concatenate([s, mean, mx, mn], axis=1)
    hg = hg.reshape(NG, EMB, 4)
    hg = jnp.transpose(hg, (0, 2, 1))
    batch_mask = jnp.ones((NG, 4), dtype=bool)
    return (hg, batch_mask)


# speedup vs baseline: 10.2498x; 1.0575x over previous
import functools

import jax
import jax.numpy as jnp
from jax import lax
from jax.experimental import pallas as pl
from jax.experimental.pallas import tpu as pltpu
from jax.experimental.pallas import tpu_sc as plsc

N = 10000
E = 160000
EMB = 128
NUM_LAYER = 3
NG = 64

NS = 16
NP = 10240
RPS = NP // NS
CHUNK = 100
NCHUNKS = 100
NBUF = 3
GRP = 4
NGRP = NCHUNKS // GRP

_mesh = plsc.VectorSubcoreMesh(core_axis_name="c", subcore_axis_name="s")


def _seg_body(h_hbm, ep_hbm, ei_hbm, z_hbm, idxr, bufr, acc, gsem, ssem, isem):
    cid = lax.axis_index("c")
    sid = lax.axis_index("s")
    r0 = sid * RPS

    def fire_idx_group(g, slot):
        base = sid * NCHUNKS + g * GRP

        @pl.when(cid == 0)
        def _():
            pltpu.async_copy(ep_hbm.at[0, pl.ds(base, GRP)], idxr.at[slot, 0],
                             isem.at[slot])
            pltpu.async_copy(ep_hbm.at[1, pl.ds(base, GRP)], idxr.at[slot, 1],
                             isem.at[slot])

        @pl.when(cid == 1)
        def _():
            pltpu.async_copy(ei_hbm.at[0, pl.ds(base, GRP)], idxr.at[slot, 0],
                             isem.at[slot])
            pltpu.async_copy(ei_hbm.at[1, pl.ds(base, GRP)], idxr.at[slot, 1],
                             isem.at[slot])

    def wait_idx_group(g, slot):
        base = sid * NCHUNKS + g * GRP
        pltpu.make_async_copy(ep_hbm.at[0, pl.ds(base, GRP)],
                              idxr.at[slot, 0], isem.at[slot]).wait()
        pltpu.make_async_copy(ep_hbm.at[1, pl.ds(base, GRP)],
                              idxr.at[slot, 1], isem.at[slot]).wait()

    pltpu.sync_copy(h_hbm.at[pl.ds(r0, RPS)], acc.at[pl.ds(r0, RPS)])

    plsc.subcore_barrier()

    fire_idx_group(0, 0)
    wait_idx_group(0, 0)

    def fire_pro(c, carry):
        pltpu.async_copy(h_hbm.at[idxr.at[0, 0, c]], bufr.at[c], gsem.at[c])
        return carry

    lax.fori_loop(0, 2, fire_pro, 0)

    def body(ch, carry):
        b = lax.rem(ch, NBUF)
        g = lax.div(ch, GRP)
        s = lax.rem(g, 2)
        k = lax.rem(ch, GRP)
        pltpu.make_async_copy(h_hbm.at[idxr.at[s, 0, k]], bufr.at[b],
                              gsem.at[b]).wait()
        pltpu.async_copy(bufr.at[b], acc.at[idxr.at[s, 1, k]], ssem.at[b],
                         add=True)
        f = ch + 2

        @pl.when(f < NCHUNKS)
        def _():
            bf = lax.rem(f, NBUF)
            gf = lax.div(f, GRP)
            sf = lax.rem(gf, 2)
            kf = lax.rem(f, GRP)

            @pl.when(f >= NBUF)
            def _():
                fo = f - NBUF
                so = lax.rem(lax.div(fo, GRP), 2)
                ko = lax.rem(fo, GRP)
                pltpu.make_async_copy(bufr.at[bf], acc.at[idxr.at[so, 1, ko]],
                                      ssem.at[bf]).wait()

            @pl.when(kf == 0)
            def _():
                wait_idx_group(gf, sf)

            pltpu.async_copy(h_hbm.at[idxr.at[sf, 0, kf]], bufr.at[bf],
                             gsem.at[bf])

        @pl.when(jnp.logical_and(k == 0, ch + GRP < NCHUNKS))
        def _():
            fire_idx_group(g + 1, lax.rem(g + 1, 2))

        return carry

    lax.fori_loop(0, NCHUNKS, body, 0)

    def drain(ch, carry):
        b = lax.rem(ch, NBUF)
        s = lax.rem(lax.div(ch, GRP), 2)
        k = lax.rem(ch, GRP)
        pltpu.make_async_copy(bufr.at[b], acc.at[idxr.at[s, 1, k]],
                              ssem.at[b]).wait()
        return carry

    lax.fori_loop(NCHUNKS - NBUF, NCHUNKS, drain, 0)

    plsc.subcore_barrier()
    pltpu.sync_copy(acc.at[pl.ds(r0, RPS)], z_hbm.at[cid, pl.ds(r0, RPS)])


_seg_call = pl.kernel(
    _seg_body,
    out_type=[jax.ShapeDtypeStruct((2, NP, EMB), jnp.float32)],
    mesh=_mesh,
    scratch_types=[
        pltpu.VMEM((2, 2, GRP, CHUNK), jnp.int32),
        pltpu.VMEM((NBUF, CHUNK, EMB), jnp.float32),
        pltpu.VMEM_SHARED((NP, EMB), jnp.float32),
        pltpu.SemaphoreType.DMA((NBUF,)),
        pltpu.SemaphoreType.DMA((NBUF,)),
        pltpu.SemaphoreType.DMA((2,)),
    ],
)


def _embed_body(x_ref, te_ref, oe_ref, out_ref):
    xb = x_ref[...]
    x0 = xb[:, 0:1].astype(jnp.float32)
    x1 = xb[:, 1:2].astype(jnp.float32)
    t0 = te_ref[0:1, :]
    t1 = te_ref[1:2, :]
    o0 = oe_ref[0:1, :]
    o1 = oe_ref[1:2, :]
    h = t0 + (t1 - t0) * x0 + o0 + (o1 - o0) * x1
    out_ref[:N, :] = h
    out_ref[N:, :] = jnp.zeros((NP - N, EMB), jnp.float32)


def _embed_call(x, type_emb, out_emb):
    return pl.pallas_call(
        _embed_body,
        out_shape=jax.ShapeDtypeStruct((NP, EMB), jnp.float32),
    )(x, type_emb, out_emb)


def _layer_body(zp_ref, zi_ref, w1p, b1p, w2p, b2p, w1i, b1i, w2i, b2i,
                out_ref, *, final):
    zp = zp_ref[0]
    a = jnp.maximum(
        jnp.dot(zp, w1p[...], preferred_element_type=jnp.float32) + b1p[...], 0.0)
    hp = jnp.dot(a, w2p[...], preferred_element_type=jnp.float32) + b2p[...]
    zi = zi_ref[0]
    b = jnp.maximum(
        jnp.dot(zi, w1i[...], preferred_element_type=jnp.float32) + b1i[...], 0.0)
    hi = jnp.dot(b, w2i[...], preferred_element_type=jnp.float32) + b2i[...]
    o = hp + hi
    if not final:
        o = jnp.maximum(o, 0.0)
    out_ref[...] = o


def _layer_call(z2, weights, final):
    BN = 1024
    zpspec = pl.BlockSpec((1, BN, EMB), lambda i: (0, i, 0))
    zispec = pl.BlockSpec((1, BN, EMB), lambda i: (1, i, 0))
    wspec = pl.BlockSpec((EMB, EMB), lambda i: (0, 0))
    bias = pl.BlockSpec((1, EMB), lambda i: (0, 0))
    return pl.pallas_call(
        functools.partial(_layer_body, final=final),
        grid=(NP // BN,),
        in_specs=[zpspec, zispec, wspec, bias, wspec, bias,
                  wspec, bias, wspec, bias],
        out_specs=pl.BlockSpec((BN, EMB), lambda i: (i, 0)),
        out_shape=jax.ShapeDtypeStruct((NP, EMB), jnp.float32),
    )(z2, z2, *weights)


AGG_BN = 1024
AGG_NBLK = NP // AGG_BN


def _final_body(zp_ref, zi_ref, w1p, b1p, w2p, b2p, w1i, b1i, w2i, b2i,
                bcol_ref, brow_ref, s_out, mean_out, mx_out, mn_out,
                s_acc, c_acc, mx_acc, mn_acc):
    i = pl.program_id(0)

    @pl.when(i == 0)
    def _():
        s_acc[...] = jnp.zeros((NG, EMB), jnp.float32)
        c_acc[...] = jnp.zeros((NG, EMB), jnp.float32)
        mx_acc[...] = jnp.full((NG, EMB), -jnp.inf, jnp.float32)
        mn_acc[...] = jnp.full((NG, EMB), jnp.inf, jnp.float32)

    zp = zp_ref[0]
    a = jnp.maximum(
        jnp.dot(zp, w1p[...], preferred_element_type=jnp.float32) + b1p[...], 0.0)
    hp = jnp.dot(a, w2p[...], preferred_element_type=jnp.float32) + b2p[...]
    zi = zi_ref[0]
    b = jnp.maximum(
        jnp.dot(zi, w1i[...], preferred_element_type=jnp.float32) + b1i[...], 0.0)
    hi = jnp.dot(b, w2i[...], preferred_element_type=jnp.float32) + b2i[...]
    hb = hp + hi

    bcol = bcol_ref[...]
    brow = brow_ref[...]

    onehot_t = (lax.broadcasted_iota(jnp.int32, (NG, AGG_BN), 0)
                == brow).astype(jnp.float32)
    s_acc[...] += jnp.dot(onehot_t, hb, preferred_element_type=jnp.float32)
    c_acc[...] += jnp.broadcast_to(
        jnp.sum(onehot_t, axis=1, keepdims=True), (NG, EMB))

    g_lo = jnp.min(bcol)
    g_hi = jnp.max(bcol)

    def gbody(g, carry):
        m = bcol == g
        mxr = jnp.max(jnp.where(m, hb, -jnp.inf), axis=0, keepdims=True)
        mnr = jnp.min(jnp.where(m, hb, jnp.inf), axis=0, keepdims=True)
        sel = lax.broadcasted_iota(jnp.int32, (NG, 1), 0) == g
        mx_acc[...] = jnp.where(sel, jnp.maximum(mx_acc[...], mxr), mx_acc[...])
        mn_acc[...] = jnp.where(sel, jnp.minimum(mn_acc[...], mnr), mn_acc[...])
        return carry

    lax.fori_loop(g_lo, g_hi + 1, gbody, 0)

    @pl.when(i == AGG_NBLK - 1)
    def _():
        s = s_acc[...]
        s_out[...] = s
        mean_out[...] = s / jnp.maximum(c_acc[...], 1.0)
        mx_out[...] = mx_acc[...]
        mn_out[...] = mn_acc[...]


def _final_call(z2, weights, batch_col, batch_row):
    zpspec = pl.BlockSpec((1, AGG_BN, EMB), lambda i: (0, i, 0))
    zispec = pl.BlockSpec((1, AGG_BN, EMB), lambda i: (1, i, 0))
    wspec = pl.BlockSpec((EMB, EMB), lambda i: (0, 0))
    bias = pl.BlockSpec((1, EMB), lambda i: (0, 0))
    ospec = pl.BlockSpec((NG, EMB), lambda i: (0, 0))
    osd = jax.ShapeDtypeStruct((NG, EMB), jnp.float32)
    return pl.pallas_call(
        _final_body,
        grid=(AGG_NBLK,),
        in_specs=[zpspec, zispec, wspec, bias, wspec, bias,
                  wspec, bias, wspec, bias,
                  pl.BlockSpec((AGG_BN, 1), lambda i: (i, 0)),
                  pl.BlockSpec((1, AGG_BN), lambda i: (0, i))],
        out_specs=[ospec, ospec, ospec, ospec],
        out_shape=[osd, osd, osd, osd],
        scratch_shapes=[pltpu.VMEM((NG, EMB), jnp.float32)] * 4,
    )(z2, z2, *weights, batch_col, batch_row)


def kernel(x, edge_index_pos, edge_index_inv, batch, type_emb, out_emb,
           W1, b1, W2, b2):
    ep3 = edge_index_pos.reshape(2, NS * NCHUNKS, CHUNK)
    ei3 = edge_index_inv.reshape(2, NS * NCHUNKS, CHUNK)

    bpad = jnp.concatenate([batch, jnp.full((NP - N,), NG, jnp.int32)])

    h = _embed_call(x, type_emb, out_emb)
    for l in range(NUM_LAYER):
        z2 = _seg_call(h, ep3, ei3)
        if isinstance(z2, (list, tuple)):
            z2 = z2[0]
        weights = (W1[l, 0], b1[l, 0].reshape(1, EMB),
                   W2[l, 0], b2[l, 0].reshape(1, EMB),
                   W1[l, 1], b1[l, 1].reshape(1, EMB),
                   W2[l, 1], b2[l, 1].reshape(1, EMB))
        if l < NUM_LAYER - 1:
            h = _layer_call(z2, weights, final=False)
        else:
            s, mean, mx, mn = _final_call(
                z2, weights, bpad.reshape(NP, 1), bpad.reshape(1, NP))

    hg = jnp.concatenate([s, mean, mx, mn], axis=1)
    hg = hg.reshape(NG, EMB, 4)
    hg = jnp.transpose(hg, (0, 2, 1))
    batch_mask = jnp.ones((NG, 4), dtype=bool)
    return (hg, batch_mask)
